# baseline (speedup 1.0000x reference)
import functools

import jax
import jax.numpy as jnp
from jax import lax
from jax.experimental import pallas as pl
from jax.experimental.pallas import tpu as pltpu
from jax.experimental.pallas import tpu_sc as plsc

B = 4096
L = 200
V = 10000
D = 512
H = 256
NC, NS = 2, 16
NW = NC * NS
RPW = B // NW
UPR = 5
QTR = L // UPR
NUNIT = RPW * UPR
LANES = 16
NCHUNK = H // LANES


def _pool_body(ids_hbm, table_hbm, out_hbm, idx_v, gbuf, accv, gsem, osem):
    c = lax.axis_index("c")
    s = lax.axis_index("s")
    wid = s * NC + c
    base = wid * RPW

    pltpu.sync_copy(ids_hbm.at[pl.ds(base * L, RPW * L)], idx_v)

    def _gather(u, bi):
        return pltpu.make_async_copy(
            table_hbm.at[idx_v.at[pl.ds(u * QTR, QTR)]], gbuf.at[bi],
            gsem.at[bi])

    _gather(0, 0).start()
    _gather(1, 1).start()

    inv = jnp.float32(1.0 / L)

    def row_body(r, carry):
        a = r % 2

        @pl.when(r >= 2)
        def _():
            pltpu.make_async_copy(
                accv.at[a], out_hbm.at[base + r - 2], osem.at[a]).wait()

        u0 = r * UPR
        acc = tuple(jnp.zeros((LANES,), jnp.float32) for _ in range(NCHUNK))
        for h in range(UPR):
            @pl.when(u0 + h + 2 < NUNIT)
            def _(h=h):
                _gather(u0 + h + 2, (h + 2) % UPR).start()

            _gather(u0 + h, h).wait()

            def kbody(k8, acc, h=h):
                for kk in range(8):
                    k = k8 * 8 + kk
                    acc = tuple(
                        acc[j] + gbuf[h, k, pl.ds(j * LANES, LANES)]
                        for j in range(NCHUNK))
                return acc

            acc = lax.fori_loop(0, QTR // 8, kbody, acc)

        for j in range(NCHUNK):
            accv[a, pl.ds(j * LANES, LANES)] = acc[j] * inv
        pltpu.async_copy(accv.at[a], out_hbm.at[base + r], osem.at[a])
        return carry

    lax.fori_loop(0, RPW, row_body, 0)

    pltpu.make_async_copy(
        accv.at[0], out_hbm.at[base + RPW - 2], osem.at[0]).wait()
    pltpu.make_async_copy(
        accv.at[1], out_hbm.at[base + RPW - 1], osem.at[1]).wait()


@functools.cache
def _pool():
    return pl.kernel(
        _pool_body,
        out_type=jax.ShapeDtypeStruct((B, H), jnp.float32),
        mesh=plsc.VectorSubcoreMesh(
            core_axis_name="c", subcore_axis_name="s",
            num_cores=NC, num_subcores=NS),
        scratch_types=[
            pltpu.VMEM((RPW * L,), jnp.int32),
            pltpu.VMEM((UPR, QTR, H), jnp.float32),
            pltpu.VMEM((2, H), jnp.float32),
            pltpu.SemaphoreType.DMA((UPR,)),
            pltpu.SemaphoreType.DMA((2,)),
        ],
    )


_PROJ_BLK = 1000


def _proj_body(t_ref, w_ref, o_ref):
    o_ref[...] = jnp.dot(
        t_ref[...], w_ref[...], preferred_element_type=jnp.float32)


def _proj(emb_table, W1):
    return pl.pallas_call(
        _proj_body,
        grid=(V // _PROJ_BLK,),
        in_specs=[
            pl.BlockSpec((_PROJ_BLK, D), lambda i: (i, 0)),
            pl.BlockSpec((D, H), lambda i: (0, 0)),
        ],
        out_specs=pl.BlockSpec((_PROJ_BLK, H), lambda i: (i, 0)),
        out_shape=jax.ShapeDtypeStruct((V, H), jnp.float32),
    )(emb_table, W1)


def _mlp_body(p_ref, b1_ref, w2_ref, b2_ref, f_ref, a_ref):
    h = jnp.maximum(p_ref[...] + b1_ref[...], 0.0)
    f = jnp.dot(h, w2_ref[...], preferred_element_type=jnp.float32) + b2_ref[...]
    f_ref[...] = f
    m = jnp.max(f, axis=1, keepdims=True)
    e = jnp.exp(f - m)
    a_ref[...] = e / jnp.sum(e, axis=1, keepdims=True)


_MLP_BLK = 1024


def _mlp(pooled1, b1, W2, b2):
    return pl.pallas_call(
        _mlp_body,
        grid=(B // _MLP_BLK,),
        in_specs=[
            pl.BlockSpec((_MLP_BLK, H), lambda i: (i, 0)),
            pl.BlockSpec((1, H), lambda i: (0, 0)),
            pl.BlockSpec((H, H), lambda i: (0, 0)),
            pl.BlockSpec((1, H), lambda i: (0, 0)),
        ],
        out_specs=[
            pl.BlockSpec((_MLP_BLK, H), lambda i: (i, 0)),
            pl.BlockSpec((_MLP_BLK, H), lambda i: (i, 0)),
        ],
        out_shape=[
            jax.ShapeDtypeStruct((B, H), jnp.float32),
            jax.ShapeDtypeStruct((B, H), jnp.float32),
        ],
    )(pooled1, b1.reshape(1, H), W2, b2.reshape(1, H))


def kernel(input_ids, attention_mask, emb_table, W1, b1, W2, b2):
    ids = input_ids.astype(jnp.int32).reshape(B * L)
    t1 = _proj(emb_table, W1)
    pooled1 = _pool()(ids, t1)
    features, attention_weights = _mlp(pooled1, b1, W2, b2)
    return features, attention_weights


# speedup vs baseline: 1.4400x; 1.4400x over previous
import functools

import jax
import jax.numpy as jnp
from jax import lax
from jax.experimental import pallas as pl
from jax.experimental.pallas import tpu as pltpu
from jax.experimental.pallas import tpu_sc as plsc

B = 4096
L = 200
V = 10000
D = 512
H = 256
NC, NS = 2, 16
NW = NC * NS
RPW = B // NW
UPR = 5
QTR = L // UPR
NUNIT = RPW * UPR
LANES = 16
NCHUNK = H // LANES


def _pool_body(ids_hbm, table_hbm, out_hbm, idx_v, gbuf, accv, gsem, osem):
    c = lax.axis_index("c")
    s = lax.axis_index("s")
    wid = s * NC + c
    base = wid * RPW

    pltpu.sync_copy(ids_hbm.at[pl.ds(base * L, RPW * L)], idx_v)

    def _gather(u, bi):
        return pltpu.make_async_copy(
            table_hbm.at[idx_v.at[pl.ds(u * QTR, QTR)]], gbuf.at[bi],
            gsem.at[bi])

    _gather(0, 0).start()
    _gather(1, 1).start()
    _gather(2, 2).start()

    inv = jnp.float32(1.0 / L)

    def row_body(r, carry):
        a = r % 2

        @pl.when(r >= 2)
        def _():
            pltpu.make_async_copy(
                accv.at[a], out_hbm.at[base + r - 2], osem.at[a]).wait()

        u0 = r * UPR
        acc = tuple(jnp.zeros((LANES,), jnp.float32) for _ in range(NCHUNK))
        for h in range(UPR):
            @pl.when(u0 + h + 3 < NUNIT)
            def _(h=h):
                _gather(u0 + h + 3, (h + 3) % UPR).start()

            _gather(u0 + h, h).wait()

            def kbody(k4, acc, h=h):
                for kk in range(4):
                    k = k4 * 4 + kk
                    acc = tuple(
                        acc[j] + gbuf[h, k, pl.ds(j * LANES, LANES)]
                        for j in range(NCHUNK))
                return acc

            acc = lax.fori_loop(0, QTR // 4, kbody, acc)

        for j in range(NCHUNK):
            accv[a, pl.ds(j * LANES, LANES)] = acc[j] * inv
        pltpu.async_copy(accv.at[a], out_hbm.at[base + r], osem.at[a])
        return carry

    lax.fori_loop(0, RPW, row_body, 0)

    pltpu.make_async_copy(
        accv.at[0], out_hbm.at[base + RPW - 2], osem.at[0]).wait()
    pltpu.make_async_copy(
        accv.at[1], out_hbm.at[base + RPW - 1], osem.at[1]).wait()


@functools.cache
def _pool():
    return pl.kernel(
        _pool_body,
        out_type=jax.ShapeDtypeStruct((B, H), jnp.float32),
        mesh=plsc.VectorSubcoreMesh(
            core_axis_name="c", subcore_axis_name="s",
            num_cores=NC, num_subcores=NS),
        scratch_types=[
            pltpu.VMEM((RPW * L,), jnp.int32),
            pltpu.VMEM((UPR, QTR, H), jnp.float32),
            pltpu.VMEM((2, H), jnp.float32),
            pltpu.SemaphoreType.DMA((UPR,)),
            pltpu.SemaphoreType.DMA((2,)),
        ],
    )


_PROJ_BLK = 1000


def _proj_body(t_ref, w_ref, o_ref):
    o_ref[...] = jnp.dot(
        t_ref[...], w_ref[...], preferred_element_type=jnp.float32)


def _proj(emb_table, W1):
    return pl.pallas_call(
        _proj_body,
        grid=(V // _PROJ_BLK,),
        in_specs=[
            pl.BlockSpec((_PROJ_BLK, D), lambda i: (i, 0)),
            pl.BlockSpec((D, H), lambda i: (0, 0)),
        ],
        out_specs=pl.BlockSpec((_PROJ_BLK, H), lambda i: (i, 0)),
        out_shape=jax.ShapeDtypeStruct((V, H), jnp.float32),
    )(emb_table, W1)


def _mlp_body(p_ref, b1_ref, w2_ref, b2_ref, f_ref, a_ref):
    h = jnp.maximum(p_ref[...] + b1_ref[...], 0.0)
    f = jnp.dot(h, w2_ref[...], preferred_element_type=jnp.float32) + b2_ref[...]
    f_ref[...] = f
    m = jnp.max(f, axis=1, keepdims=True)
    e = jnp.exp(f - m)
    a_ref[...] = e / jnp.sum(e, axis=1, keepdims=True)


_MLP_BLK = 1024


def _mlp(pooled1, b1, W2, b2):
    return pl.pallas_call(
        _mlp_body,
        grid=(B // _MLP_BLK,),
        in_specs=[
            pl.BlockSpec((_MLP_BLK, H), lambda i: (i, 0)),
            pl.BlockSpec((1, H), lambda i: (0, 0)),
            pl.BlockSpec((H, H), lambda i: (0, 0)),
            pl.BlockSpec((1, H), lambda i: (0, 0)),
        ],
        out_specs=[
            pl.BlockSpec((_MLP_BLK, H), lambda i: (i, 0)),
            pl.BlockSpec((_MLP_BLK, H), lambda i: (i, 0)),
        ],
        out_shape=[
            jax.ShapeDtypeStruct((B, H), jnp.float32),
            jax.ShapeDtypeStruct((B, H), jnp.float32),
        ],
    )(pooled1, b1.reshape(1, H), W2, b2.reshape(1, H))


def kernel(input_ids, attention_mask, emb_table, W1, b1, W2, b2):
    ids = input_ids.astype(jnp.int32).reshape(B * L)
    t1 = _proj(emb_table, W1)
    pooled1 = _pool()(ids, t1)
    features, attention_weights = _mlp(pooled1, b1, W2, b2)
    return features, attention_weights


# speedup vs baseline: 1.5054x; 1.0454x over previous
import functools

import jax
import jax.numpy as jnp
from jax import lax
from jax.experimental import pallas as pl
from jax.experimental.pallas import tpu as pltpu
from jax.experimental.pallas import tpu_sc as plsc

B = 4096
L = 200
V = 10000
D = 512
H = 256
NC, NS = 2, 16
NW = NC * NS
RPW = B // NW
UPR = 5
QTR = L // UPR
NUNIT = RPW * UPR
LANES = 16
NCHUNK = H // LANES


def _pool_body(ids_hbm, table_hbm, out_hbm, idx_v, gbuf, accv, gsem, osem):
    c = lax.axis_index("c")
    s = lax.axis_index("s")
    wid = s * NC + c
    base = wid * RPW

    pltpu.sync_copy(ids_hbm.at[pl.ds(base * L, RPW * L)], idx_v)

    def _gather(u, bi):
        return pltpu.make_async_copy(
            table_hbm.at[idx_v.at[pl.ds(u * QTR, QTR)]], gbuf.at[bi],
            gsem.at[bi])

    _gather(0, 0).start()
    _gather(1, 1).start()
    _gather(2, 2).start()
    _gather(3, 3).start()

    inv = jnp.float32(1.0 / L)

    def row_body(r, carry):
        a = r % 2

        @pl.when(r >= 2)
        def _():
            pltpu.make_async_copy(
                accv.at[a], out_hbm.at[base + r - 2], osem.at[a]).wait()

        u0 = r * UPR
        acc = tuple(jnp.zeros((LANES,), jnp.float32) for _ in range(NCHUNK))
        for h in range(UPR):
            @pl.when(u0 + h + 4 < NUNIT)
            def _(h=h):
                _gather(u0 + h + 4, (h + 4) % UPR).start()

            _gather(u0 + h, h).wait()

            def kbody(k4, acc, h=h):
                for kk in range(4):
                    k = k4 * 4 + kk
                    acc = tuple(
                        acc[j] + gbuf[h, k, pl.ds(j * LANES, LANES)]
                        for j in range(NCHUNK))
                return acc

            acc = lax.fori_loop(0, QTR // 4, kbody, acc)

        for j in range(NCHUNK):
            accv[a, pl.ds(j * LANES, LANES)] = acc[j] * inv
        pltpu.async_copy(accv.at[a], out_hbm.at[base + r], osem.at[a])
        return carry

    lax.fori_loop(0, RPW, row_body, 0)

    pltpu.make_async_copy(
        accv.at[0], out_hbm.at[base + RPW - 2], osem.at[0]).wait()
    pltpu.make_async_copy(
        accv.at[1], out_hbm.at[base + RPW - 1], osem.at[1]).wait()


@functools.cache
def _pool():
    return pl.kernel(
        _pool_body,
        out_type=jax.ShapeDtypeStruct((B, H), jnp.float32),
        mesh=plsc.VectorSubcoreMesh(
            core_axis_name="c", subcore_axis_name="s",
            num_cores=NC, num_subcores=NS),
        scratch_types=[
            pltpu.VMEM((RPW * L,), jnp.int32),
            pltpu.VMEM((UPR, QTR, H), jnp.float32),
            pltpu.VMEM((2, H), jnp.float32),
            pltpu.SemaphoreType.DMA((UPR,)),
            pltpu.SemaphoreType.DMA((2,)),
        ],
    )


_PROJ_BLK = 1000


def _proj_body(t_ref, w_ref, o_ref):
    o_ref[...] = jnp.dot(
        t_ref[...], w_ref[...], preferred_element_type=jnp.float32)


def _proj(emb_table, W1):
    return pl.pallas_call(
        _proj_body,
        grid=(V // _PROJ_BLK,),
        in_specs=[
            pl.BlockSpec((_PROJ_BLK, D), lambda i: (i, 0)),
            pl.BlockSpec((D, H), lambda i: (0, 0)),
        ],
        out_specs=pl.BlockSpec((_PROJ_BLK, H), lambda i: (i, 0)),
        out_shape=jax.ShapeDtypeStruct((V, H), jnp.float32),
    )(emb_table, W1)


def _mlp_body(p_ref, b1_ref, w2_ref, b2_ref, f_ref, a_ref):
    h = jnp.maximum(p_ref[...] + b1_ref[...], 0.0)
    f = jnp.dot(h, w2_ref[...], preferred_element_type=jnp.float32) + b2_ref[...]
    f_ref[...] = f
    m = jnp.max(f, axis=1, keepdims=True)
    e = jnp.exp(f - m)
    a_ref[...] = e / jnp.sum(e, axis=1, keepdims=True)


_MLP_BLK = 1024


def _mlp(pooled1, b1, W2, b2):
    return pl.pallas_call(
        _mlp_body,
        grid=(B // _MLP_BLK,),
        in_specs=[
            pl.BlockSpec((_MLP_BLK, H), lambda i: (i, 0)),
            pl.BlockSpec((1, H), lambda i: (0, 0)),
            pl.BlockSpec((H, H), lambda i: (0, 0)),
            pl.BlockSpec((1, H), lambda i: (0, 0)),
        ],
        out_specs=[
            pl.BlockSpec((_MLP_BLK, H), lambda i: (i, 0)),
            pl.BlockSpec((_MLP_BLK, H), lambda i: (i, 0)),
        ],
        out_shape=[
            jax.ShapeDtypeStruct((B, H), jnp.float32),
            jax.ShapeDtypeStruct((B, H), jnp.float32),
        ],
    )(pooled1, b1.reshape(1, H), W2, b2.reshape(1, H))


def kernel(input_ids, attention_mask, emb_table, W1, b1, W2, b2):
    ids = input_ids.astype(jnp.int32).reshape(B * L)
    t1 = _proj(emb_table, W1)
    pooled1 = _pool()(ids, t1)
    features, attention_weights = _mlp(pooled1, b1, W2, b2)
    return features, attention_weights
